# Initial kernel scaffold; baseline (speedup 1.0000x reference)
import jax
import jax.numpy as jnp
from jax.experimental import pallas as pl


def kernel(x, gate_w, gate_b, W1, b1, W2, b2):
    raise NotImplementedError("write your pallas kernel here")


# speedup vs baseline: 1.3858x; 1.3858x over previous
import functools

import jax
import jax.numpy as jnp
from jax import lax
from jax.experimental import pallas as pl
from jax.experimental.pallas import tpu as pltpu
from jax.experimental.pallas import tpu_sc as plsc

_E = 8
_K = 2
_DIN = 768
_DH = 3072
_DOUT = 768
_N = 2048

_TM = 128
_NPAIR = _N * _K
_P = 5120
_G = _P // _TM

_NC = 2
_NS = 16
_NW = _NC * _NS
_CP = _NPAIR // _NW
_CT = _N // _NW


def _router_kernel(x_ref, gw_ref, gb_ref,
                   probs_ref, aux_ref, tw_ref, pos_ref, eot_ref):
    x = x_ref[:]
    logits = jnp.dot(x, gw_ref[:], preferred_element_type=jnp.float32) + gb_ref[:]
    m = jnp.max(logits, axis=1, keepdims=True)
    ex = jnp.exp(logits - m)
    probs = ex / jnp.sum(ex, axis=1, keepdims=True)
    probs_ref[:] = probs

    mp = jnp.mean(probs, axis=0, keepdims=True)
    aux_ref[:] = jnp.sum(mp * jnp.log(mp * _E + 1e-10), axis=1, keepdims=True)

    ii = lax.broadcasted_iota(jnp.int32, (_N, _E), 1)
    v1 = jnp.max(probs, axis=1, keepdims=True)
    i1 = jnp.min(jnp.where(probs >= v1, ii, _E), axis=1, keepdims=True)
    oh1 = ii == i1
    pm = jnp.where(oh1, -1.0, probs)
    v2 = jnp.max(pm, axis=1, keepdims=True)
    i2 = jnp.min(jnp.where(pm >= v2, ii, _E), axis=1, keepdims=True)
    oh2 = ii == i2
    den = v1 + v2 + 1e-10
    wa = v1 / den
    wb = v2 / den
    tw_ref[:] = jnp.concatenate(
        [jnp.broadcast_to(wa[:, :, None], (_N, 1, 16)),
         jnp.broadcast_to(wb[:, :, None], (_N, 1, 16))], axis=1)

    cnt = oh1.astype(jnp.float32) + oh2.astype(jnp.float32)
    nb = _N // 128
    r = lax.broadcasted_iota(jnp.int32, (128, 128), 0)
    c = lax.broadcasted_iota(jnp.int32, (128, 128), 1)
    tril = (c < r).astype(jnp.float32)
    blocks, sums = [], []
    for b in range(nb):
        blk = cnt[b * 128:(b + 1) * 128, :]
        blocks.append(jnp.dot(tril, blk, preferred_element_type=jnp.float32))
        sums.append(jnp.sum(blk, axis=0, keepdims=True))
    s = jnp.concatenate(sums, axis=0)
    r2 = lax.broadcasted_iota(jnp.int32, (nb, nb), 0)
    c2 = lax.broadcasted_iota(jnp.int32, (nb, nb), 1)
    tril2 = (c2 < r2).astype(jnp.float32)
    carry = jnp.dot(tril2, s, preferred_element_type=jnp.float32)
    cex = jnp.concatenate(
        [blocks[b] + carry[b:b + 1, :] for b in range(nb)], axis=0)

    tot = jnp.sum(s, axis=0, keepdims=True)
    cpad = (tot.astype(jnp.int32) + (_TM - 1)) // _TM * _TM
    r3 = lax.broadcasted_iota(jnp.int32, (_E, _E), 0)
    c3 = lax.broadcasted_iota(jnp.int32, (_E, _E), 1)
    sup = (r3 < c3).astype(jnp.float32)
    off = jnp.dot(cpad.astype(jnp.float32), sup,
                  preferred_element_type=jnp.float32)

    base = off + cex
    pa = jnp.sum(jnp.where(oh1, base, 0.0), axis=1, keepdims=True)
    pb = jnp.sum(jnp.where(oh2, base, 0.0), axis=1, keepdims=True)
    pos_ref[:] = jnp.concatenate(
        [pa.astype(jnp.int32), pb.astype(jnp.int32)], axis=1)

    tv = (lax.broadcasted_iota(jnp.int32, (_G, _E), 0) * _TM).astype(jnp.float32)
    eot = jnp.sum((off <= tv).astype(jnp.float32), axis=1, keepdims=True) - 1.0
    eot_ref[:] = eot.astype(jnp.int32)


_router = pl.pallas_call(
    _router_kernel,
    out_shape=[
        jax.ShapeDtypeStruct((_N, _E), jnp.float32),
        jax.ShapeDtypeStruct((1, 1), jnp.float32),
        jax.ShapeDtypeStruct((_N, _K, 16), jnp.float32),
        jax.ShapeDtypeStruct((_N, _K), jnp.int32),
        jax.ShapeDtypeStruct((_G, 1), jnp.int32),
    ],
)


@functools.cache
def _sc_mesh():
    return plsc.VectorSubcoreMesh(core_axis_name="c", subcore_axis_name="s")


def _dispatch_body(x_hbm, pa_hbm, pb_hbm, xs_hbm,
                   pa_v, pb_v, rows_v, sem1, sem2):
    wid = lax.axis_index("s") * _NC + lax.axis_index("c")
    bt = wid * _CT
    pltpu.sync_copy(pa_hbm.at[pl.ds(bt, _CT)], pa_v)
    pltpu.sync_copy(pb_hbm.at[pl.ds(bt, _CT)], pb_v)
    pltpu.sync_copy(x_hbm.at[pl.ds(bt, _CT)], rows_v)
    c1 = pltpu.async_copy(rows_v, xs_hbm.at[pa_v], sem1)
    c2 = pltpu.async_copy(rows_v, xs_hbm.at[pb_v], sem2)
    c1.wait()
    c2.wait()


@functools.cache
def _dispatch():
    return pl.kernel(
        _dispatch_body,
        out_type=jax.ShapeDtypeStruct((_P, _DIN), jnp.float32),
        mesh=_sc_mesh(),
        scratch_types=[
            pltpu.VMEM((_CT,), jnp.int32),
            pltpu.VMEM((_CT,), jnp.int32),
            pltpu.VMEM((_CT, _DIN), jnp.float32),
            pltpu.SemaphoreType.DMA,
            pltpu.SemaphoreType.DMA,
        ],
    )


def _ffn_kernel(eot_ref, x_ref, w1_ref, b1_ref, w2_ref, b2_ref, o_ref):
    del eot_ref
    h = jnp.dot(x_ref[:], w1_ref[0], preferred_element_type=jnp.float32)
    h = jnp.maximum(h + b1_ref[0], 0.0)
    o = jnp.dot(h, w2_ref[0], preferred_element_type=jnp.float32) + b2_ref[0]
    o_ref[:] = o


_ffn = pl.pallas_call(
    _ffn_kernel,
    grid_spec=pltpu.PrefetchScalarGridSpec(
        num_scalar_prefetch=1,
        grid=(_G,),
        in_specs=[
            pl.BlockSpec((_TM, _DIN), lambda t, eot: (t, 0)),
            pl.BlockSpec((1, _DIN, _DH), lambda t, eot: (eot[t], 0, 0)),
            pl.BlockSpec((1, 1, _DH), lambda t, eot: (eot[t], 0, 0)),
            pl.BlockSpec((1, _DH, _DOUT), lambda t, eot: (eot[t], 0, 0)),
            pl.BlockSpec((1, 1, _DOUT), lambda t, eot: (eot[t], 0, 0)),
        ],
        out_specs=pl.BlockSpec((_TM, _DOUT), lambda t, eot: (t, 0)),
    ),
    out_shape=jax.ShapeDtypeStruct((_P, _DOUT), jnp.float32),
    compiler_params=pltpu.CompilerParams(
        dimension_semantics=("arbitrary",)),
)


def _combine_body(p_hbm, pa_hbm, pb_hbm, twa_hbm, twb_hbm, out_hbm,
                  pa_v, pb_v, twa_v, twb_v, a_v, b_v, sem1, sem2):
    wid = lax.axis_index("s") * _NC + lax.axis_index("c")
    bt = wid * _CT
    pltpu.sync_copy(pa_hbm.at[pl.ds(bt, _CT)], pa_v)
    pltpu.sync_copy(pb_hbm.at[pl.ds(bt, _CT)], pb_v)
    pltpu.sync_copy(twa_hbm.at[pl.ds(bt, _CT)], twa_v)
    pltpu.sync_copy(twb_hbm.at[pl.ds(bt, _CT)], twb_v)
    ca = pltpu.async_copy(p_hbm.at[pa_v], a_v, sem1)
    cb = pltpu.async_copy(p_hbm.at[pb_v], b_v, sem2)
    ca.wait()
    cb.wait()

    def body(t, carry):
        wa = twa_v[t, :]
        wb = twb_v[t, :]
        for c in range(_DOUT // 16):
            sl = pl.ds(c * 16, 16)
            a_v[t, sl] = wa * a_v[t, sl] + wb * b_v[t, sl]
        return carry

    lax.fori_loop(0, _CT, body, 0)
    pltpu.sync_copy(a_v, out_hbm.at[pl.ds(bt, _CT)])


@functools.cache
def _combine():
    return pl.kernel(
        _combine_body,
        out_type=jax.ShapeDtypeStruct((_N, _DOUT), jnp.float32),
        mesh=_sc_mesh(),
        scratch_types=[
            pltpu.VMEM((_CT,), jnp.int32),
            pltpu.VMEM((_CT,), jnp.int32),
            pltpu.VMEM((_CT, 16), jnp.float32),
            pltpu.VMEM((_CT, 16), jnp.float32),
            pltpu.VMEM((_CT, _DOUT), jnp.float32),
            pltpu.VMEM((_CT, _DOUT), jnp.float32),
            pltpu.SemaphoreType.DMA,
            pltpu.SemaphoreType.DMA,
        ],
    )


def kernel(x, gate_w, gate_b, W1, b1, W2, b2):
    probs, aux, tw3, pos2, eot2 = _router(x, gate_w, gate_b.reshape(1, _E))
    twa = tw3[:, 0, :]
    twb = tw3[:, 1, :]
    pos_a = pos2[:, 0]
    pos_b = pos2[:, 1]
    eot = eot2.reshape(_G)
    xs = _dispatch()(x, pos_a, pos_b)
    pairs_out = _ffn(eot, xs, W1, b1.reshape(_E, 1, _DH),
                     W2, b2.reshape(_E, 1, _DOUT))
    out = _combine()(pairs_out, pos_a, pos_b, twa, twb)
    return out, aux.reshape(()), probs
